# baseline (speedup 1.0000x reference)
import functools

import jax
import jax.numpy as jnp
from jax import lax
from jax.experimental import pallas as pl
from jax.experimental.pallas import tpu as pltpu
from jax.experimental.pallas import tpu_sc as plsc

NUM_BUCKETS = 1000000
D = 32
ROWS, COLS = 16384, 26
B = ROWS * COLS
NC, NS, L = 2, 16, 16
NW = NC * NS
BPW = B // NW
IW = 128
NBLK = BPW // IW
NPAR = 4

FULL_COLS = NUM_BUCKETS // 128
TAIL = NUM_BUCKETS - FULL_COLS * 128
R_ROWS = NUM_BUCKETS // 4
PAIRS = FULL_COLS // 2
PAIR_BASE = PAIRS // NW
PAIR_EXTRA = PAIRS - PAIR_BASE * NW

_SC_PARAMS = pltpu.CompilerParams(
    use_tc_tiling_on_sc=True, needs_layout_passes=False)


def _relayout_body(tab_hbm, tail_hbm, tre_hbm, vi0, vi1, vo0, vo1, cst_v,
                   is0, is1, os0, os1):
    vins = (vi0, vi1)
    vouts = (vo0, vo1)
    isems = (is0, is1)
    osems = (os0, os1)
    wid = lax.axis_index("s") * NC + lax.axis_index("c")
    npairs = jnp.where(wid < jnp.int32(PAIR_EXTRA),
                       jnp.int32(PAIR_BASE + 1), jnp.int32(PAIR_BASE))
    base = (wid * jnp.int32(PAIR_BASE)
            + lax.min(wid, jnp.int32(PAIR_EXTRA))) * jnp.int32(2)

    iota = lax.iota(jnp.int32, L)
    for g in range(8):
        lv = jnp.int32(16 * g) + iota
        cst_v[g, :] = lax.bitwise_and(lv, jnp.int32(3)) * jnp.int32(32)
        cst_v[8 + g, :] = lax.shift_right_logical(lv, jnp.int32(2))

    def col_slice(col):
        lane0 = pl.multiple_of(col * jnp.int32(128), 128)
        return tab_hbm.at[:, pl.ds(lane0, 128)]

    def fire_in(col, par):
        return pltpu.async_copy(col_slice(col), vins[par], isems[par])

    def wait_in(par):
        pltpu.make_async_copy(col_slice(jnp.int32(0)), vins[par],
                              isems[par]).wait()

    def out_slice(col):
        r0 = pl.multiple_of(col * jnp.int32(32), 32)
        return tre_hbm.at[pl.ds(r0, 32)]

    def fire_out(col, par):
        return pltpu.async_copy(vouts[par], out_slice(col), osems[par])

    def wait_out(par):
        pltpu.make_async_copy(vouts[par], out_slice(jnp.int32(0)),
                              osems[par]).wait()

    def transpose(par):
        vin = vins[par]
        vout = vouts[par]
        for g in range(8):
            lanev = cst_v[g, :]
            rowv = cst_v[8 + g, :]
            for c in range(32):
                vals = vin[c, pl.ds(g * L, L)]
                plsc.store_scatter(vout, [rowv, lanev + jnp.int32(c)], vals)

    fire_in(base, 0)
    fire_in(base + jnp.int32(1), 1)

    def step(pp, carry):
        for par in range(2):
            col = base + pp * jnp.int32(2) + jnp.int32(par)
            wait_in(par)

            @pl.when(pp >= 1)
            def _():
                wait_out(par)

            transpose(par)
            fire_out(col, par)

            @pl.when(pp + 1 < npairs)
            def _():
                fire_in(col + jnp.int32(2), par)
        return carry

    lax.fori_loop(jnp.int32(0), npairs, step, 0)
    wait_out(0)
    wait_out(1)

    @pl.when(wid == jnp.int32(0))
    def _():
        pltpu.sync_copy(tail_hbm, vi0.at[pl.ds(0, 16)])
        pltpu.sync_copy(vi0.at[pl.ds(0, 16)],
                        tre_hbm.at[pl.ds(R_ROWS - 16, 16)])


@jax.jit
def _sc_relayout(table_t, tail16):
    mesh = plsc.VectorSubcoreMesh(core_axis_name="c", subcore_axis_name="s")
    kfn = pl.kernel(
        _relayout_body,
        out_type=jax.ShapeDtypeStruct((R_ROWS, 128), jnp.float32),
        mesh=mesh,
        compiler_params=_SC_PARAMS,
        scratch_types=[
            pltpu.VMEM((32, 128), jnp.float32),
            pltpu.VMEM((32, 128), jnp.float32),
            pltpu.VMEM((32, 128), jnp.float32),
            pltpu.VMEM((32, 128), jnp.float32),
            pltpu.VMEM((16, L), jnp.int32),
            pltpu.SemaphoreType.DMA,
            pltpu.SemaphoreType.DMA,
            pltpu.SemaphoreType.DMA,
            pltpu.SemaphoreType.DMA,
        ],
    )
    return kfn(table_t, tail16)


def _gather_body(ids_hbm, trelay_hbm, out_hbm, idx_v, sub_v,
                 gb0, gb1, gb2, gb3, ob0, ob1, ob2, ob3, cst_v,
                 gs0, gs1, gs2, gs3, os0, os1, os2, os3):
    gbufs = (gb0, gb1, gb2, gb3)
    obufs = (ob0, ob1, ob2, ob3)
    gsems = (gs0, gs1, gs2, gs3)
    osems = (os0, os1, os2, os3)
    wid = lax.axis_index("s") * NC + lax.axis_index("c")
    qbase = wid * BPW

    pltpu.sync_copy(ids_hbm.at[pl.ds(wid * NBLK, NBLK)], idx_v)

    iota = lax.iota(jnp.int32, L)
    cst_v[0, :] = iota

    def hash_row(j, carry):
        c10 = jnp.int32(10)
        c1023 = jnp.int32(1023)
        cm1 = jnp.int32(219264)
        cm2 = jnp.int32(435761)
        cmod = jnp.int32(NUM_BUCKETS)
        for g in range(IW // L):
            v = idx_v[j, pl.ds(g * L, L)]
            a = lax.shift_right_logical(v, c10)
            b = lax.bitwise_and(v, c1023)
            h = lax.rem(a * cm1 + b * cm2, cmod)
            idx_v[j, pl.ds(g * L, L)] = lax.shift_right_logical(
                h, jnp.int32(2))
            sub_v[j, pl.ds(g * L, L)] = lax.shift_left(
                lax.bitwise_and(h, jnp.int32(3)), jnp.int32(5))
        return carry

    lax.fori_loop(jnp.int32(0), jnp.int32(NBLK), hash_row, 0)

    def fire_gather(j, par):
        return pltpu.async_copy(
            trelay_hbm.at[idx_v.at[j]], gbufs[par], gsems[par])

    def wait_gather(par):
        pltpu.make_async_copy(
            trelay_hbm.at[idx_v.at[jnp.int32(0)]], gbufs[par],
            gsems[par]).wait()

    def out_slice(j):
        qb = qbase + j * IW
        c = lax.shift_right_logical(qb, jnp.int32(14))
        r = pl.multiple_of(lax.bitwise_and(qb, jnp.int32(16383)), IW)
        return out_hbm.at[c, :, pl.ds(r, IW)]

    def fire_out(j, par):
        return pltpu.async_copy(obufs[par], out_slice(j), osems[par])

    def wait_out(par):
        pltpu.make_async_copy(obufs[par], out_slice(jnp.int32(0)),
                              osems[par]).wait()

    def extract(j, par):
        gbuf = gbufs[par]
        obuf = obufs[par]
        iota_v = cst_v[0, :]
        for g in range(IW // L):
            rowv = jnp.int32(g * L) + iota_v
            subv = sub_v[j, pl.ds(g * L, L)]
            for d in range(D):
                vals = plsc.load_gather(gbuf, [rowv, subv + jnp.int32(d)])
                obuf[d, pl.ds(g * L, L)] = vals

    for par in range(NPAR):
        fire_gather(jnp.int32(par), par)

    def step(jj, carry):
        for par in range(NPAR):
            j = jj * jnp.int32(NPAR) + jnp.int32(par)
            wait_gather(par)

            @pl.when(jj >= 1)
            def _():
                wait_out(par)

            extract(j, par)
            fire_out(j, par)

            @pl.when(jj <= NBLK // NPAR - 2)
            def _():
                fire_gather(j + jnp.int32(NPAR), par)
        return carry

    lax.fori_loop(jnp.int32(0), jnp.int32(NBLK // NPAR), step, 0)
    for par in range(NPAR):
        wait_out(par)


@jax.jit
def _sc_gather(ids2d, trelay):
    mesh = plsc.VectorSubcoreMesh(core_axis_name="c", subcore_axis_name="s")
    kfn = pl.kernel(
        _gather_body,
        out_type=jax.ShapeDtypeStruct((COLS, D, ROWS), jnp.float32),
        mesh=mesh,
        compiler_params=_SC_PARAMS,
        scratch_types=[
            pltpu.VMEM((NBLK, IW), jnp.int32),
            pltpu.VMEM((NBLK, IW), jnp.int32),
            pltpu.VMEM((IW, 128), jnp.float32),
            pltpu.VMEM((IW, 128), jnp.float32),
            pltpu.VMEM((IW, 128), jnp.float32),
            pltpu.VMEM((IW, 128), jnp.float32),
            pltpu.VMEM((D, IW), jnp.float32),
            pltpu.VMEM((D, IW), jnp.float32),
            pltpu.VMEM((D, IW), jnp.float32),
            pltpu.VMEM((D, IW), jnp.float32),
            pltpu.VMEM((1, L), jnp.int32),
            pltpu.SemaphoreType.DMA,
            pltpu.SemaphoreType.DMA,
            pltpu.SemaphoreType.DMA,
            pltpu.SemaphoreType.DMA,
            pltpu.SemaphoreType.DMA,
            pltpu.SemaphoreType.DMA,
            pltpu.SemaphoreType.DMA,
            pltpu.SemaphoreType.DMA,
        ],
    )
    return kfn(ids2d, trelay)


def kernel(input_ids, table):
    ids2d = jnp.transpose(input_ids).astype(jnp.int32).reshape(B // IW, IW)
    table_t = jnp.transpose(table)
    tail16 = table[FULL_COLS * 128:].reshape(16, 128)
    trelay = _sc_relayout(table_t, tail16)
    out_t = _sc_gather(ids2d, trelay)
    return jnp.transpose(out_t, (2, 0, 1))


# speedup vs baseline: 1.0217x; 1.0217x over previous
import functools

import jax
import jax.numpy as jnp
from jax import lax
from jax.experimental import pallas as pl
from jax.experimental.pallas import tpu as pltpu
from jax.experimental.pallas import tpu_sc as plsc

NUM_BUCKETS = 1000000
D = 32
ROWS, COLS = 16384, 26
B = ROWS * COLS
NC, NS, L = 2, 16, 16
NW = NC * NS
BPW = B // NW
IW = 128
NBLK = BPW // IW
NPAR = 4

FULL_COLS = NUM_BUCKETS // 128
TAIL = NUM_BUCKETS - FULL_COLS * 128
R_ROWS = NUM_BUCKETS // 4
GC = 4
GROUPS = FULL_COLS // GC
GRP_BASE = GROUPS // NW
GRP_EXTRA = GROUPS - GRP_BASE * NW

_SC_PARAMS = pltpu.CompilerParams(
    use_tc_tiling_on_sc=True, needs_layout_passes=False)


def _relayout_body(tab_hbm, tail_hbm, tre_hbm, vi0, vi1, vo0, vo1, cst_v,
                   is0, is1, os0, os1):
    vins = (vi0, vi1)
    vouts = (vo0, vo1)
    isems = (is0, is1)
    osems = (os0, os1)
    wid = lax.axis_index("s") * NC + lax.axis_index("c")
    ngrp = jnp.where(wid < jnp.int32(GRP_EXTRA),
                     jnp.int32(GRP_BASE + 1), jnp.int32(GRP_BASE))
    base = wid * jnp.int32(GRP_BASE) + lax.min(wid, jnp.int32(GRP_EXTRA))

    iota = lax.iota(jnp.int32, L)
    for g in range(8):
        lv = jnp.int32(16 * g) + iota
        cst_v[g, :] = lax.bitwise_and(lv, jnp.int32(3)) * jnp.int32(32)
        cst_v[8 + g, :] = lax.shift_right_logical(lv, jnp.int32(2))

    def in_slice(grp):
        lane0 = pl.multiple_of(grp * jnp.int32(GC * 128), 128)
        return tab_hbm.at[:, pl.ds(lane0, GC * 128)]

    def fire_in(grp, par):
        return pltpu.async_copy(in_slice(grp), vins[par], isems[par])

    def wait_in(par):
        pltpu.make_async_copy(in_slice(jnp.int32(0)), vins[par],
                              isems[par]).wait()

    def out_slice(grp):
        r0 = pl.multiple_of(grp * jnp.int32(GC * 32), 32)
        return tre_hbm.at[pl.ds(r0, GC * 32)]

    def fire_out(grp, par):
        return pltpu.async_copy(vouts[par], out_slice(grp), osems[par])

    def wait_out(par):
        pltpu.make_async_copy(vouts[par], out_slice(jnp.int32(0)),
                              osems[par]).wait()

    def transpose(par):
        vin = vins[par]
        vout = vouts[par]
        for g in range(8):
            lanev = cst_v[g, :]
            rowbase = cst_v[8 + g, :]
            for k in range(GC):
                rowv = rowbase + jnp.int32(k * 32)
                for c in range(32):
                    vals = vin[c, pl.ds(k * 128 + g * L, L)]
                    plsc.store_scatter(
                        vout, [rowv, lanev + jnp.int32(c)], vals)

    fire_in(base, 0)
    fire_in(base + jnp.int32(1), 1)
    limit = base + ngrp

    def step(pp, carry):
        for par in range(2):
            grp = base + pp * jnp.int32(2) + jnp.int32(par)

            @pl.when(grp < limit)
            def _():
                wait_in(par)

                @pl.when(pp >= 1)
                def _():
                    wait_out(par)

                transpose(par)
                fire_out(grp, par)

                @pl.when(grp + jnp.int32(2) < limit)
                def _():
                    fire_in(grp + jnp.int32(2), par)
        return carry

    niter = (ngrp + jnp.int32(1)) // jnp.int32(2)
    lax.fori_loop(jnp.int32(0), niter, step, 0)
    wait_out(0)
    wait_out(1)

    @pl.when(wid == jnp.int32(0))
    def _():
        pltpu.sync_copy(tail_hbm, vo0.at[pl.ds(0, 16)])
        pltpu.sync_copy(vo0.at[pl.ds(0, 16)],
                        tre_hbm.at[pl.ds(R_ROWS - 16, 16)])


@jax.jit
def _sc_relayout(table_t, tail16):
    mesh = plsc.VectorSubcoreMesh(core_axis_name="c", subcore_axis_name="s")
    kfn = pl.kernel(
        _relayout_body,
        out_type=jax.ShapeDtypeStruct((R_ROWS, 128), jnp.float32),
        mesh=mesh,
        compiler_params=_SC_PARAMS,
        scratch_types=[
            pltpu.VMEM((32, GC * 128), jnp.float32),
            pltpu.VMEM((32, GC * 128), jnp.float32),
            pltpu.VMEM((GC * 32, 128), jnp.float32),
            pltpu.VMEM((GC * 32, 128), jnp.float32),
            pltpu.VMEM((16, L), jnp.int32),
            pltpu.SemaphoreType.DMA,
            pltpu.SemaphoreType.DMA,
            pltpu.SemaphoreType.DMA,
            pltpu.SemaphoreType.DMA,
        ],
    )
    return kfn(table_t, tail16)


def _gather_body(ids_hbm, trelay_hbm, out_hbm, idx_v, sub_v,
                 gb0, gb1, gb2, gb3, ob0, ob1, ob2, ob3, cst_v,
                 gs0, gs1, gs2, gs3, os0, os1, os2, os3):
    gbufs = (gb0, gb1, gb2, gb3)
    obufs = (ob0, ob1, ob2, ob3)
    gsems = (gs0, gs1, gs2, gs3)
    osems = (os0, os1, os2, os3)
    wid = lax.axis_index("s") * NC + lax.axis_index("c")
    qbase = wid * BPW

    pltpu.sync_copy(ids_hbm.at[pl.ds(wid * NBLK, NBLK)], idx_v)

    iota = lax.iota(jnp.int32, L)
    cst_v[0, :] = iota

    def hash_row(j, carry):
        c10 = jnp.int32(10)
        c1023 = jnp.int32(1023)
        cm1 = jnp.int32(219264)
        cm2 = jnp.int32(435761)
        cmod = jnp.int32(NUM_BUCKETS)
        for g in range(IW // L):
            v = idx_v[j, pl.ds(g * L, L)]
            a = lax.shift_right_logical(v, c10)
            b = lax.bitwise_and(v, c1023)
            h = lax.rem(a * cm1 + b * cm2, cmod)
            idx_v[j, pl.ds(g * L, L)] = lax.shift_right_logical(
                h, jnp.int32(2))
            sub_v[j, pl.ds(g * L, L)] = lax.shift_left(
                lax.bitwise_and(h, jnp.int32(3)), jnp.int32(5))
        return carry

    lax.fori_loop(jnp.int32(0), jnp.int32(NBLK), hash_row, 0)

    def fire_gather(j, par):
        return pltpu.async_copy(
            trelay_hbm.at[idx_v.at[j]], gbufs[par], gsems[par])

    def wait_gather(par):
        pltpu.make_async_copy(
            trelay_hbm.at[idx_v.at[jnp.int32(0)]], gbufs[par],
            gsems[par]).wait()

    def out_slice(j):
        qb = qbase + j * IW
        c = lax.shift_right_logical(qb, jnp.int32(14))
        r = pl.multiple_of(lax.bitwise_and(qb, jnp.int32(16383)), IW)
        return out_hbm.at[c, :, pl.ds(r, IW)]

    def fire_out(j, par):
        return pltpu.async_copy(obufs[par], out_slice(j), osems[par])

    def wait_out(par):
        pltpu.make_async_copy(obufs[par], out_slice(jnp.int32(0)),
                              osems[par]).wait()

    def extract(j, par):
        gbuf = gbufs[par]
        obuf = obufs[par]
        iota_v = cst_v[0, :]
        for g in range(IW // L):
            rowv = jnp.int32(g * L) + iota_v
            subv = sub_v[j, pl.ds(g * L, L)]
            for d in range(D):
                vals = plsc.load_gather(gbuf, [rowv, subv + jnp.int32(d)])
                obuf[d, pl.ds(g * L, L)] = vals

    for par in range(NPAR):
        fire_gather(jnp.int32(par), par)

    def step(jj, carry):
        for par in range(NPAR):
            j = jj * jnp.int32(NPAR) + jnp.int32(par)
            wait_gather(par)

            @pl.when(jj >= 1)
            def _():
                wait_out(par)

            extract(j, par)
            fire_out(j, par)

            @pl.when(jj <= NBLK // NPAR - 2)
            def _():
                fire_gather(j + jnp.int32(NPAR), par)
        return carry

    lax.fori_loop(jnp.int32(0), jnp.int32(NBLK // NPAR), step, 0)
    for par in range(NPAR):
        wait_out(par)


@jax.jit
def _sc_gather(ids2d, trelay):
    mesh = plsc.VectorSubcoreMesh(core_axis_name="c", subcore_axis_name="s")
    kfn = pl.kernel(
        _gather_body,
        out_type=jax.ShapeDtypeStruct((COLS, D, ROWS), jnp.float32),
        mesh=mesh,
        compiler_params=_SC_PARAMS,
        scratch_types=[
            pltpu.VMEM((NBLK, IW), jnp.int32),
            pltpu.VMEM((NBLK, IW), jnp.int32),
            pltpu.VMEM((IW, 128), jnp.float32),
            pltpu.VMEM((IW, 128), jnp.float32),
            pltpu.VMEM((IW, 128), jnp.float32),
            pltpu.VMEM((IW, 128), jnp.float32),
            pltpu.VMEM((D, IW), jnp.float32),
            pltpu.VMEM((D, IW), jnp.float32),
            pltpu.VMEM((D, IW), jnp.float32),
            pltpu.VMEM((D, IW), jnp.float32),
            pltpu.VMEM((1, L), jnp.int32),
            pltpu.SemaphoreType.DMA,
            pltpu.SemaphoreType.DMA,
            pltpu.SemaphoreType.DMA,
            pltpu.SemaphoreType.DMA,
            pltpu.SemaphoreType.DMA,
            pltpu.SemaphoreType.DMA,
            pltpu.SemaphoreType.DMA,
            pltpu.SemaphoreType.DMA,
        ],
    )
    return kfn(ids2d, trelay)


def kernel(input_ids, table):
    ids2d = jnp.transpose(input_ids).astype(jnp.int32).reshape(B // IW, IW)
    table_t = jnp.transpose(table)
    tail16 = table[FULL_COLS * 128:].reshape(16, 128)
    trelay = _sc_relayout(table_t, tail16)
    out_t = _sc_gather(ids2d, trelay)
    return jnp.transpose(out_t, (2, 0, 1))


# speedup vs baseline: 1.6468x; 1.6117x over previous
import functools

import jax
import jax.numpy as jnp
from jax import lax
from jax.experimental import pallas as pl
from jax.experimental.pallas import tpu as pltpu
from jax.experimental.pallas import tpu_sc as plsc

NUM_BUCKETS = 1000000
D = 32
ROWS, COLS = 16384, 26
B = ROWS * COLS
NC, NS, L = 2, 16, 16
NW = NC * NS
BPW = B // NW
IW = 128
NBLK = BPW // IW
NPAR = 4

TCL = 8192
Q = TCL // 4
TC_GRID = (NUM_BUCKETS + TCL - 1) // TCL
R_ROWS = TC_GRID * Q

_SC_PARAMS = pltpu.CompilerParams(
    use_tc_tiling_on_sc=True, needs_layout_passes=False)


def _relayout_body(x_ref, o_ref):
    for q in range(4):
        o_ref[:, q * 32:(q + 1) * 32] = jnp.transpose(
            x_ref[:, q * Q:(q + 1) * Q], (1, 0))


def _tc_relayout(table_t):
    return pl.pallas_call(
        _relayout_body,
        grid=(TC_GRID,),
        in_specs=[pl.BlockSpec((32, TCL), lambda k: (jnp.int32(0), k))],
        out_specs=pl.BlockSpec((Q, 128), lambda k: (k, jnp.int32(0))),
        out_shape=jax.ShapeDtypeStruct((R_ROWS, 128), jnp.float32),
    )(table_t)


def _gather_body(ids_hbm, trelay_hbm, out_hbm, idx_v, sub_v,
                 gb0, gb1, gb2, gb3, ob0, ob1, ob2, ob3, cst_v,
                 gs0, gs1, gs2, gs3, os0, os1, os2, os3):
    gbufs = (gb0, gb1, gb2, gb3)
    obufs = (ob0, ob1, ob2, ob3)
    gsems = (gs0, gs1, gs2, gs3)
    osems = (os0, os1, os2, os3)
    wid = lax.axis_index("s") * NC + lax.axis_index("c")
    qbase = wid * BPW

    pltpu.sync_copy(ids_hbm.at[pl.ds(wid * NBLK, NBLK)], idx_v)

    iota = lax.iota(jnp.int32, L)
    cst_v[0, :] = iota

    lt = jnp.int32(TCL.bit_length() - 1)
    lq = jnp.int32(Q.bit_length() - 1)
    qm = jnp.int32(Q - 1)

    def hash_row(j, carry):
        c10 = jnp.int32(10)
        c1023 = jnp.int32(1023)
        cm1 = jnp.int32(219264)
        cm2 = jnp.int32(435761)
        cmod = jnp.int32(NUM_BUCKETS)
        for g in range(IW // L):
            v = idx_v[j, pl.ds(g * L, L)]
            a = lax.shift_right_logical(v, c10)
            b = lax.bitwise_and(v, c1023)
            h = lax.rem(a * cm1 + b * cm2, cmod)
            idx_v[j, pl.ds(g * L, L)] = lax.bitwise_or(
                lax.shift_left(lax.shift_right_logical(h, lt), lq),
                lax.bitwise_and(h, qm))
            sub_v[j, pl.ds(g * L, L)] = lax.shift_left(
                lax.bitwise_and(lax.shift_right_logical(h, lq), jnp.int32(3)),
                jnp.int32(5))
        return carry

    lax.fori_loop(jnp.int32(0), jnp.int32(NBLK), hash_row, 0)

    def fire_gather(j, par):
        return pltpu.async_copy(
            trelay_hbm.at[idx_v.at[j]], gbufs[par], gsems[par])

    def wait_gather(par):
        pltpu.make_async_copy(
            trelay_hbm.at[idx_v.at[jnp.int32(0)]], gbufs[par],
            gsems[par]).wait()

    def out_slice(j):
        qb = qbase + j * IW
        c = lax.shift_right_logical(qb, jnp.int32(14))
        r = pl.multiple_of(lax.bitwise_and(qb, jnp.int32(16383)), IW)
        return out_hbm.at[c, :, pl.ds(r, IW)]

    def fire_out(j, par):
        return pltpu.async_copy(obufs[par], out_slice(j), osems[par])

    def wait_out(par):
        pltpu.make_async_copy(obufs[par], out_slice(jnp.int32(0)),
                              osems[par]).wait()

    def extract(j, par):
        gbuf = gbufs[par]
        obuf = obufs[par]
        iota_v = cst_v[0, :]
        for g in range(IW // L):
            rowv = jnp.int32(g * L) + iota_v
            subv = sub_v[j, pl.ds(g * L, L)]
            for d in range(D):
                vals = plsc.load_gather(gbuf, [rowv, subv + jnp.int32(d)])
                obuf[d, pl.ds(g * L, L)] = vals

    for par in range(NPAR):
        fire_gather(jnp.int32(par), par)

    def step(jj, carry):
        for par in range(NPAR):
            j = jj * jnp.int32(NPAR) + jnp.int32(par)
            wait_gather(par)

            @pl.when(jj >= 1)
            def _():
                wait_out(par)

            extract(j, par)
            fire_out(j, par)

            @pl.when(jj <= NBLK // NPAR - 2)
            def _():
                fire_gather(j + jnp.int32(NPAR), par)
        return carry

    lax.fori_loop(jnp.int32(0), jnp.int32(NBLK // NPAR), step, 0)
    for par in range(NPAR):
        wait_out(par)


@jax.jit
def _sc_gather(ids2d, trelay):
    mesh = plsc.VectorSubcoreMesh(core_axis_name="c", subcore_axis_name="s")
    kfn = pl.kernel(
        _gather_body,
        out_type=jax.ShapeDtypeStruct((COLS, D, ROWS), jnp.float32),
        mesh=mesh,
        compiler_params=_SC_PARAMS,
        scratch_types=[
            pltpu.VMEM((NBLK, IW), jnp.int32),
            pltpu.VMEM((NBLK, IW), jnp.int32),
            pltpu.VMEM((IW, 128), jnp.float32),
            pltpu.VMEM((IW, 128), jnp.float32),
            pltpu.VMEM((IW, 128), jnp.float32),
            pltpu.VMEM((IW, 128), jnp.float32),
            pltpu.VMEM((D, IW), jnp.float32),
            pltpu.VMEM((D, IW), jnp.float32),
            pltpu.VMEM((D, IW), jnp.float32),
            pltpu.VMEM((D, IW), jnp.float32),
            pltpu.VMEM((1, L), jnp.int32),
            pltpu.SemaphoreType.DMA,
            pltpu.SemaphoreType.DMA,
            pltpu.SemaphoreType.DMA,
            pltpu.SemaphoreType.DMA,
            pltpu.SemaphoreType.DMA,
            pltpu.SemaphoreType.DMA,
            pltpu.SemaphoreType.DMA,
            pltpu.SemaphoreType.DMA,
        ],
    )
    return kfn(ids2d, trelay)


def kernel(input_ids, table):
    ids2d = jnp.transpose(input_ids).astype(jnp.int32).reshape(B // IW, IW)
    table_t = jnp.transpose(table)
    trelay = _tc_relayout(table_t)
    out_t = _sc_gather(ids2d, trelay)
    return jnp.transpose(out_t, (2, 0, 1))


# speedup vs baseline: 1.8466x; 1.1214x over previous
import functools

import jax
import jax.numpy as jnp
from jax import lax
from jax.experimental import pallas as pl
from jax.experimental.pallas import tpu as pltpu
from jax.experimental.pallas import tpu_sc as plsc

NUM_BUCKETS = 1000000
D = 32
ROWS, COLS = 16384, 26
B = ROWS * COLS
NC, NS, L = 2, 16, 16
NW = NC * NS
BPW = B // NW
IW = 128
NBLK = BPW // IW
NPAR = 2

TCL = 8192
Q = TCL // 4
TC_GRID = (NUM_BUCKETS + TCL - 1) // TCL
R_ROWS = TC_GRID * Q

_SC_PARAMS = pltpu.CompilerParams(
    use_tc_tiling_on_sc=True, needs_layout_passes=False)


def _relayout_body(x_ref, o_ref):
    for q in range(4):
        o_ref[:, q * 32:(q + 1) * 32] = jnp.transpose(
            x_ref[:, q * Q:(q + 1) * Q], (1, 0))


def _tc_relayout(table_t):
    return pl.pallas_call(
        _relayout_body,
        grid=(TC_GRID,),
        in_specs=[pl.BlockSpec((32, TCL), lambda k: (jnp.int32(0), k))],
        out_specs=pl.BlockSpec((Q, 128), lambda k: (k, jnp.int32(0))),
        out_shape=jax.ShapeDtypeStruct((R_ROWS, 128), jnp.float32),
    )(table_t)


def _gather_body(ids_hbm, trelay_hbm, out_hbm, idx_v, sub_v,
                 gb0, gb1, ob0, ob1, mb0, mb1, cst_v,
                 gs0, gs1, os0, os1):
    gbufs = (gb0, gb1)
    obufs = (ob0, ob1)
    mbufs = (mb0, mb1)
    gsems = (gs0, gs1)
    osems = (os0, os1)
    wid = lax.axis_index("s") * NC + lax.axis_index("c")
    qbase = wid * BPW

    pltpu.sync_copy(ids_hbm.at[pl.ds(wid * NBLK, NBLK)], idx_v)

    iota = lax.iota(jnp.int32, L)
    cst_v[0, :] = iota

    lt = jnp.int32(TCL.bit_length() - 1)
    lq = jnp.int32(Q.bit_length() - 1)
    qm = jnp.int32(Q - 1)

    def hash_row(j, carry):
        c10 = jnp.int32(10)
        c1023 = jnp.int32(1023)
        cm1 = jnp.int32(219264)
        cm2 = jnp.int32(435761)
        cmod = jnp.int32(NUM_BUCKETS)
        for g in range(IW // L):
            v = idx_v[j, pl.ds(g * L, L)]
            a = lax.shift_right_logical(v, c10)
            b = lax.bitwise_and(v, c1023)
            h = lax.rem(a * cm1 + b * cm2, cmod)
            idx_v[j, pl.ds(g * L, L)] = lax.bitwise_or(
                lax.shift_left(lax.shift_right_logical(h, lt), lq),
                lax.bitwise_and(h, qm))
            sub_v[j, pl.ds(g * L, L)] = lax.shift_left(
                lax.bitwise_and(lax.shift_right_logical(h, lq), jnp.int32(3)),
                jnp.int32(5))
        return carry

    lax.fori_loop(jnp.int32(0), jnp.int32(NBLK), hash_row, 0)

    def fire_gather(j, par):
        return pltpu.async_copy(
            trelay_hbm.at[idx_v.at[j]], gbufs[par], gsems[par])

    def wait_gather(par):
        pltpu.make_async_copy(
            trelay_hbm.at[idx_v.at[jnp.int32(0)]], gbufs[par],
            gsems[par]).wait()

    def out_slice(j):
        qb = qbase + j * IW
        c = lax.shift_right_logical(qb, jnp.int32(14))
        r = pl.multiple_of(lax.bitwise_and(qb, jnp.int32(16383)), IW)
        return out_hbm.at[c, :, pl.ds(r, IW)]

    def fire_out(j, par):
        return pltpu.async_copy(obufs[par], out_slice(j), osems[par])

    def wait_out(par):
        pltpu.make_async_copy(obufs[par], out_slice(jnp.int32(0)),
                              osems[par]).wait()

    def extract(j, par):
        gbuf = gbufs[par]
        obuf = obufs[par]
        mid = mbufs[par]
        iota_v = cst_v[0, :]
        c31 = jnp.int32(31)

        def s1(g64, carry):
            g16 = g64 * jnp.int32(L)
            subvec = sub_v[j, pl.ds(g16, L)]
            for k in range(L):
                sub = subvec[k]
                l = g16 + jnp.int32(k)
                row_s = jnp.broadcast_to(l, (L,))
                lane0 = lax.bitwise_and(iota_v + l, c31)
                lane1 = lax.bitwise_and(iota_v + l + jnp.int32(L), c31)
                plsc.store_scatter(mid, [row_s, lane0],
                                   gbuf[l, pl.ds(sub, L)])
                plsc.store_scatter(mid, [row_s, lane1],
                                   gbuf[l, pl.ds(sub + jnp.int32(L), L)])
            return carry

        lax.fori_loop(jnp.int32(0), jnp.int32(IW // L), s1, 0)
        for g in range(IW // L):
            rowv = jnp.int32(g * L) + iota_v
            for d in range(D):
                colv = lax.bitwise_and(rowv + jnp.int32(d), c31)
                vals = plsc.load_gather(mid, [rowv, colv])
                obuf[d, pl.ds(g * L, L)] = vals

    for par in range(NPAR):
        fire_gather(jnp.int32(par), par)

    def step(jj, carry):
        for par in range(NPAR):
            j = jj * jnp.int32(NPAR) + jnp.int32(par)
            wait_gather(par)

            @pl.when(jj >= 1)
            def _():
                wait_out(par)

            extract(j, par)
            fire_out(j, par)

            @pl.when(jj <= NBLK // NPAR - 2)
            def _():
                fire_gather(j + jnp.int32(NPAR), par)
        return carry

    lax.fori_loop(jnp.int32(0), jnp.int32(NBLK // NPAR), step, 0)
    for par in range(NPAR):
        wait_out(par)


@jax.jit
def _sc_gather(ids2d, trelay):
    mesh = plsc.VectorSubcoreMesh(core_axis_name="c", subcore_axis_name="s")
    kfn = pl.kernel(
        _gather_body,
        out_type=jax.ShapeDtypeStruct((COLS, D, ROWS), jnp.float32),
        mesh=mesh,
        compiler_params=_SC_PARAMS,
        scratch_types=[
            pltpu.VMEM((NBLK, IW), jnp.int32),
            pltpu.VMEM((NBLK, IW), jnp.int32),
            pltpu.VMEM((IW, 128), jnp.float32),
            pltpu.VMEM((IW, 128), jnp.float32),
            pltpu.VMEM((D, IW), jnp.float32),
            pltpu.VMEM((D, IW), jnp.float32),
            pltpu.VMEM((IW, 128), jnp.float32),
            pltpu.VMEM((IW, 128), jnp.float32),
            pltpu.VMEM((1, L), jnp.int32),
            pltpu.SemaphoreType.DMA,
            pltpu.SemaphoreType.DMA,
            pltpu.SemaphoreType.DMA,
            pltpu.SemaphoreType.DMA,
        ],
    )
    return kfn(ids2d, trelay)


def kernel(input_ids, table):
    ids2d = jnp.transpose(input_ids).astype(jnp.int32).reshape(B // IW, IW)
    table_t = jnp.transpose(table)
    trelay = _tc_relayout(table_t)
    out_t = _sc_gather(ids2d, trelay)
    return jnp.transpose(out_t, (2, 0, 1))
